# Initial kernel scaffold; baseline (speedup 1.0000x reference)
import jax
import jax.numpy as jnp
from jax.experimental import pallas as pl


def kernel(x, edge_index, batch, W1, b1, W2, b2, Wfc, bfc):
    raise NotImplementedError("write your pallas kernel here")


# speedup vs baseline: 9.0791x; 9.0791x over previous
import functools

import jax
import jax.numpy as jnp
from jax import lax
from jax.experimental import pallas as pl
from jax.experimental.pallas import tpu as pltpu
from jax.experimental.pallas import tpu_sc as plsc

_N = 10000
_D = 128
_E = 320000
_ROBOTS = 32
_OUT = 2

_NP = 10240
_NW = 32
_NT = 16
_CHUNK = 64
_CPT = 160
_BLKC = 16
_NBLKI = _CPT // _BLKC
_EPAD = _NW * _CPT * _CHUNK
_RPT = _NP // _NT

_BLK = 256
_NBLK = _NP // _BLK

_f32 = jnp.float32


def _sc_deg_body(dst_hbm, out0, out1, deg_acc, dst_v, ones_v, buf_v):
    cid = lax.axis_index("c")
    sid = lax.axis_index("s")
    wid = cid * _NT + sid
    for k in range(_CHUNK // 16):
        ones_v[pl.ds(k * 16, 16)] = jnp.full((16,), 1.0, _f32)
    for k in range(_RPT // 16):
        buf_v[pl.ds(k * 16, 16)] = jnp.zeros((16,), _f32)
    pltpu.sync_copy(buf_v, deg_acc.at[pl.ds(sid * _RPT, _RPT)])
    plsc.subcore_barrier()

    for b in range(_NBLKI):
        pltpu.sync_copy(dst_hbm.at[wid].at[b], dst_v)

        def _step(k, carry):
            pltpu.sync_copy(ones_v, deg_acc.at[dst_v.at[k]], add=True)
            return carry

        lax.fori_loop(0, _BLKC, _step, 0)
    plsc.subcore_barrier()

    sl = pl.ds(sid * _RPT, _RPT)
    pltpu.sync_copy(deg_acc.at[sl], buf_v)

    @pl.when(cid == 0)
    def _():
        pltpu.sync_copy(buf_v, out0.at[sl])

    @pl.when(cid == 1)
    def _():
        pltpu.sync_copy(buf_v, out1.at[sl])


@functools.cache
def _get_sc_deg():
    return pl.kernel(
        _sc_deg_body,
        out_type=(jax.ShapeDtypeStruct((_NP,), _f32),
                  jax.ShapeDtypeStruct((_NP,), _f32)),
        mesh=plsc.VectorSubcoreMesh(core_axis_name="c", subcore_axis_name="s"),
        scratch_types=[
            pltpu.VMEM_SHARED((_NP,), _f32),
            pltpu.VMEM((_BLKC, _CHUNK), jnp.int32),
            pltpu.VMEM((_CHUNK,), _f32),
            pltpu.VMEM((_RPT,), _f32),
        ],
    )


def _sc_agg_body(hp_hbm, src_hbm, dst_hbm, zeros_hbm, out0, out1,
                 acc, src_v, dst_v, r_a, r_b, sem_a, sem_b, sem_i0, sem_i1):
    cid = lax.axis_index("c")
    sid = lax.axis_index("s")
    wid = cid * _NT + sid
    base = sid * _RPT
    pltpu.sync_copy(zeros_hbm, r_a)
    for k in range(_RPT // _CHUNK):
        pltpu.sync_copy(r_a, acc.at[pl.ds(base + k * _CHUNK, _CHUNK)])
    plsc.subcore_barrier()

    sems = (sem_i0, sem_i1)
    pltpu.async_copy(src_hbm.at[wid].at[0], src_v.at[0], sems[0])
    pltpu.async_copy(dst_hbm.at[wid].at[0], dst_v.at[0], sems[0])
    pltpu.async_copy(src_hbm.at[wid].at[1], src_v.at[1], sems[1])
    pltpu.async_copy(dst_hbm.at[wid].at[1], dst_v.at[1], sems[1])
    for b in range(_NBLKI):
        s = b % 2
        pltpu.make_async_copy(src_hbm.at[wid].at[b], src_v.at[s], sems[s]).wait()
        pltpu.make_async_copy(dst_hbm.at[wid].at[b], dst_v.at[s], sems[s]).wait()
        sv, dv = src_v.at[s], dst_v.at[s]
        pltpu.async_copy(hp_hbm.at[sv.at[0]], r_a, sem_a)
        pltpu.async_copy(hp_hbm.at[sv.at[1]], r_b, sem_b)

        def _step(i, carry):
            j = 2 * i
            pltpu.make_async_copy(hp_hbm.at[sv.at[j]], r_a, sem_a).wait()
            pltpu.sync_copy(r_a, acc.at[dv.at[j]], add=True)
            pltpu.async_copy(hp_hbm.at[sv.at[j + 2]], r_a, sem_a)
            pltpu.make_async_copy(hp_hbm.at[sv.at[j + 1]], r_b, sem_b).wait()
            pltpu.sync_copy(r_b, acc.at[dv.at[j + 1]], add=True)
            pltpu.async_copy(hp_hbm.at[sv.at[j + 3]], r_b, sem_b)
            return carry

        lax.fori_loop(0, _BLKC // 2 - 1, _step, 0)
        j = _BLKC - 2
        pltpu.make_async_copy(hp_hbm.at[sv.at[j]], r_a, sem_a).wait()
        pltpu.sync_copy(r_a, acc.at[dv.at[j]], add=True)
        pltpu.make_async_copy(hp_hbm.at[sv.at[j + 1]], r_b, sem_b).wait()
        pltpu.sync_copy(r_b, acc.at[dv.at[j + 1]], add=True)
        if b + 2 < _NBLKI:
            pltpu.async_copy(src_hbm.at[wid].at[b + 2], src_v.at[s], sems[s])
            pltpu.async_copy(dst_hbm.at[wid].at[b + 2], dst_v.at[s], sems[s])
    plsc.subcore_barrier()

    for k in range(_RPT // _CHUNK):
        sl = pl.ds(base + k * _CHUNK, _CHUNK)
        pltpu.sync_copy(acc.at[sl], r_a)

        @pl.when(cid == 0)
        def _():
            pltpu.sync_copy(r_a, out0.at[sl])

        @pl.when(cid == 1)
        def _():
            pltpu.sync_copy(r_a, out1.at[sl])


@functools.cache
def _get_sc_agg():
    return pl.kernel(
        _sc_agg_body,
        out_type=(jax.ShapeDtypeStruct((_NP, _D), _f32),
                  jax.ShapeDtypeStruct((_NP, _D), _f32)),
        mesh=plsc.VectorSubcoreMesh(core_axis_name="c", subcore_axis_name="s"),
        scratch_types=[
            pltpu.VMEM_SHARED((_NP, _D), _f32),
            pltpu.VMEM((2, _BLKC, _CHUNK), jnp.int32),
            pltpu.VMEM((2, _BLKC, _CHUNK), jnp.int32),
            pltpu.VMEM((_CHUNK, _D), _f32),
            pltpu.VMEM((_CHUNK, _D), _f32),
            pltpu.SemaphoreType.DMA,
            pltpu.SemaphoreType.DMA,
            pltpu.SemaphoreType.DMA,
            pltpu.SemaphoreType.DMA,
        ],
    )


def _mm_body(x_ref, w_ref, h_ref, cnt_ref):
    i = pl.program_id(0)
    xb = x_ref[...]
    h_ref[...] = jnp.dot(xb, w_ref[...], preferred_element_type=_f32)
    nz = jnp.logical_and(xb[:, 0:1] != 0.0, xb[:, 1:2] != 0.0)
    c = jnp.sum(nz.astype(_f32), axis=0, keepdims=True)

    @pl.when(i == 0)
    def _():
        cnt_ref[...] = c

    @pl.when(i != 0)
    def _():
        cnt_ref[...] = cnt_ref[...] + c


def _scale_body(d0_ref, d1_ref, h_ref, dinv_ref, hp_ref):
    deg = d0_ref[...] + d1_ref[...]
    dinv = lax.rsqrt(deg + 1.0)
    dinv_ref[...] = dinv
    hp_ref[...] = h_ref[...] * dinv


def _layer2_body(p0_ref, p1_ref, hp_ref, dinv_ref, b_ref, w_ref, out_ref):
    i = pl.program_id(0)
    dinv = dinv_ref[...]
    s = dinv * (p0_ref[...] + p1_ref[...] + hp_ref[...]) + b_ref[...]
    h = jnp.tanh(s)
    hp2 = jnp.dot(h, w_ref[...], preferred_element_type=_f32) * dinv
    rows = i * _BLK + lax.broadcasted_iota(jnp.int32, (_BLK, 1), 0)
    out_ref[...] = jnp.where(rows < _N, hp2, 0.0)


def _head_body(q0_ref, q1_ref, hp_ref, dinv_ref, b_ref, wfc_ref, bfc_ref,
               cnt_ref, out_ref, acc_ref):
    i = pl.program_id(0)
    s = dinv_ref[...] * (q0_ref[...] + q1_ref[...] + hp_ref[...]) + b_ref[...]
    h = jnp.tanh(s)
    rows = i * _BLK + lax.broadcasted_iota(jnp.int32, (_BLK, 1), 0)
    h = jnp.where(rows < _N, h, 0.0)
    part = jnp.sum(h, axis=0, keepdims=True)

    @pl.when(i == 0)
    def _():
        acc_ref[...] = part

    @pl.when(i != 0)
    def _():
        acc_ref[...] = acc_ref[...] + part

    @pl.when(i == _NBLK - 1)
    def _():
        pooled = acc_ref[...] * (1.0 / _N)
        vel = jnp.dot(pooled, wfc_ref[...], preferred_element_type=_f32)
        vel = vel + bfc_ref[...]
        robot = lax.broadcasted_iota(jnp.int32, (1, _ROBOTS * _OUT), 1) // _OUT
        mask = robot.astype(_f32) < cnt_ref[...]
        out_ref[...] = jnp.where(mask, vel, 0.0)


def _row_spec():
    return pl.BlockSpec((_BLK, _D), lambda i: (i, 0))


def _col_spec():
    return pl.BlockSpec((_BLK, 1), lambda i: (i, 0))


def _const_spec(shape):
    return pl.BlockSpec(shape, lambda i: (0,) * len(shape))


def _tc_mm(xp, w1):
    return pl.pallas_call(
        _mm_body,
        grid=(_NBLK,),
        in_specs=[_row_spec(), _const_spec((_D, _D))],
        out_specs=[_row_spec(), _const_spec((1, 1))],
        out_shape=[jax.ShapeDtypeStruct((_NP, _D), _f32),
                   jax.ShapeDtypeStruct((1, 1), _f32)],
    )(xp, w1)


def _tc_scale(d0, d1, h):
    return pl.pallas_call(
        _scale_body,
        grid=(_NBLK,),
        in_specs=[_col_spec(), _col_spec(), _row_spec()],
        out_specs=[_col_spec(), _row_spec()],
        out_shape=[jax.ShapeDtypeStruct((_NP, 1), _f32),
                   jax.ShapeDtypeStruct((_NP, _D), _f32)],
    )(d0, d1, h)


def _tc_layer2(p0, p1, hp, dinv, b1, w2):
    return pl.pallas_call(
        _layer2_body,
        grid=(_NBLK,),
        in_specs=[_row_spec(), _row_spec(), _row_spec(), _col_spec(),
                  _const_spec((1, _D)), _const_spec((_D, _D))],
        out_specs=_row_spec(),
        out_shape=jax.ShapeDtypeStruct((_NP, _D), _f32),
    )(p0, p1, hp, dinv, b1, w2)


def _tc_head(q0, q1, hp, dinv, b2, wfc, bfc, cnt):
    return pl.pallas_call(
        _head_body,
        grid=(_NBLK,),
        in_specs=[_row_spec(), _row_spec(), _row_spec(), _col_spec(),
                  _const_spec((1, _D)), _const_spec((_D, _ROBOTS * _OUT)),
                  _const_spec((1, _ROBOTS * _OUT)), _const_spec((1, 1))],
        out_specs=_const_spec((1, _ROBOTS * _OUT)),
        out_shape=jax.ShapeDtypeStruct((1, _ROBOTS * _OUT), _f32),
        scratch_shapes=[pltpu.VMEM((1, _D), _f32)],
    )(q0, q1, hp, dinv, b2, wfc, bfc, cnt)


def kernel(x, edge_index, batch, W1, b1, W2, b2, Wfc, bfc):
    del batch
    x = x.astype(_f32)
    ei = edge_index.astype(jnp.int32)
    pad_idx = jnp.full((_EPAD - _E,), _N, jnp.int32)
    src = jnp.concatenate([ei[0], pad_idx]).reshape(_NW, _NBLKI, _BLKC, _CHUNK)
    dst = jnp.concatenate([ei[1], pad_idx]).reshape(_NW, _NBLKI, _BLKC, _CHUNK)
    xp = jnp.concatenate([x, jnp.zeros((_NP - _N, _D), _f32)], axis=0)
    zeros2d = jnp.zeros((_CHUNK, _D), _f32)

    d0, d1 = _get_sc_deg()(dst)
    h1raw, cnt = _tc_mm(xp, W1.astype(_f32))
    dinv, hp1 = _tc_scale(d0.reshape(_NP, 1), d1.reshape(_NP, 1), h1raw)
    p0, p1 = _get_sc_agg()(hp1, src, dst, zeros2d)
    hp2 = _tc_layer2(p0, p1, hp1, dinv, b1.astype(_f32).reshape(1, _D),
                     W2.astype(_f32))
    q0, q1 = _get_sc_agg()(hp2, src, dst, zeros2d)
    out = _tc_head(q0, q1, hp2, dinv, b2.astype(_f32).reshape(1, _D),
                   Wfc.astype(_f32), bfc.astype(_f32).reshape(1, _ROBOTS * _OUT),
                   cnt)
    return out.reshape(1, _ROBOTS, _OUT)


# speedup vs baseline: 21.2019x; 2.3352x over previous
import functools

import jax
import jax.numpy as jnp
from jax import lax
from jax.experimental import pallas as pl
from jax.experimental.pallas import tpu as pltpu
from jax.experimental.pallas import tpu_sc as plsc

_N = 10000
_D = 128
_E = 320000
_ROBOTS = 32
_OUT = 2

_NP = 10240
_NW = 32
_NT = 16
_CHUNK = 64
_CPT = 160
_BLKC = 16
_NBLKI = _CPT // _BLKC
_EPAD = _NW * _CPT * _CHUNK
_RPT = _NP // _NT

_BLK = 256
_NBLK = _NP // _BLK

_f32 = jnp.float32


def _sc_deg_body(dst_hbm, out0, out1, deg_acc, dst_v, ones_v, buf_v):
    cid = lax.axis_index("c")
    sid = lax.axis_index("s")
    wid = cid * _NT + sid
    for k in range(_CHUNK // 16):
        ones_v[pl.ds(k * 16, 16)] = jnp.full((16,), 1.0, _f32)
    for k in range(_RPT // 16):
        buf_v[pl.ds(k * 16, 16)] = jnp.zeros((16,), _f32)
    pltpu.sync_copy(buf_v, deg_acc.at[pl.ds(sid * _RPT, _RPT)])
    plsc.subcore_barrier()

    for b in range(_NBLKI):
        pltpu.sync_copy(dst_hbm.at[wid].at[b], dst_v)

        def _step(k, carry):
            pltpu.sync_copy(ones_v, deg_acc.at[dst_v.at[k]], add=True)
            return carry

        lax.fori_loop(0, _BLKC, _step, 0)
    plsc.subcore_barrier()

    sl = pl.ds(sid * _RPT, _RPT)
    pltpu.sync_copy(deg_acc.at[sl], buf_v)

    @pl.when(cid == 0)
    def _():
        pltpu.sync_copy(buf_v, out0.at[sl])

    @pl.when(cid == 1)
    def _():
        pltpu.sync_copy(buf_v, out1.at[sl])


@functools.cache
def _get_sc_deg():
    return pl.kernel(
        _sc_deg_body,
        out_type=(jax.ShapeDtypeStruct((_NP,), _f32),
                  jax.ShapeDtypeStruct((_NP,), _f32)),
        mesh=plsc.VectorSubcoreMesh(core_axis_name="c", subcore_axis_name="s"),
        scratch_types=[
            pltpu.VMEM_SHARED((_NP,), _f32),
            pltpu.VMEM((_BLKC, _CHUNK), jnp.int32),
            pltpu.VMEM((_CHUNK,), _f32),
            pltpu.VMEM((_RPT,), _f32),
        ],
    )


def _sc_agg_body(hp_hbm, src_hbm, dst_hbm, zeros_hbm, out0, out1,
                 acc, src_v, dst_v, r_a, r_b, sem_a, sem_b, sem_i0, sem_i1):
    cid = lax.axis_index("c")
    sid = lax.axis_index("s")
    wid = cid * _NT + sid
    base = sid * _RPT
    pltpu.sync_copy(zeros_hbm, r_a)
    for k in range(_RPT // _CHUNK):
        pltpu.sync_copy(r_a, acc.at[pl.ds(base + k * _CHUNK, _CHUNK)])
    plsc.subcore_barrier()

    sems = (sem_i0, sem_i1)
    pltpu.async_copy(src_hbm.at[wid].at[0], src_v.at[0], sems[0])
    pltpu.async_copy(dst_hbm.at[wid].at[0], dst_v.at[0], sems[0])
    pltpu.async_copy(src_hbm.at[wid].at[1], src_v.at[1], sems[1])
    pltpu.async_copy(dst_hbm.at[wid].at[1], dst_v.at[1], sems[1])
    for b in range(_NBLKI):
        s = b % 2
        pltpu.make_async_copy(src_hbm.at[wid].at[b], src_v.at[s], sems[s]).wait()
        pltpu.make_async_copy(dst_hbm.at[wid].at[b], dst_v.at[s], sems[s]).wait()
        sv, dv = src_v.at[s], dst_v.at[s]
        pltpu.async_copy(hp_hbm.at[sv.at[0]], r_a, sem_a)
        pltpu.async_copy(hp_hbm.at[sv.at[1]], r_b, sem_b)

        def _step(i, carry):
            j = 2 * i
            pltpu.make_async_copy(hp_hbm.at[sv.at[j]], r_a, sem_a).wait()
            pltpu.sync_copy(r_a, acc.at[dv.at[j]], add=True)
            pltpu.async_copy(hp_hbm.at[sv.at[j + 2]], r_a, sem_a)
            pltpu.make_async_copy(hp_hbm.at[sv.at[j + 1]], r_b, sem_b).wait()
            pltpu.sync_copy(r_b, acc.at[dv.at[j + 1]], add=True)
            pltpu.async_copy(hp_hbm.at[sv.at[j + 3]], r_b, sem_b)
            return carry

        lax.fori_loop(0, _BLKC // 2 - 1, _step, 0)
        j = _BLKC - 2
        pltpu.make_async_copy(hp_hbm.at[sv.at[j]], r_a, sem_a).wait()
        pltpu.sync_copy(r_a, acc.at[dv.at[j]], add=True)
        pltpu.make_async_copy(hp_hbm.at[sv.at[j + 1]], r_b, sem_b).wait()
        pltpu.sync_copy(r_b, acc.at[dv.at[j + 1]], add=True)
        if b + 2 < _NBLKI:
            pltpu.async_copy(src_hbm.at[wid].at[b + 2], src_v.at[s], sems[s])
            pltpu.async_copy(dst_hbm.at[wid].at[b + 2], dst_v.at[s], sems[s])
    plsc.subcore_barrier()

    for k in range(_RPT // _CHUNK):
        sl = pl.ds(base + k * _CHUNK, _CHUNK)
        pltpu.sync_copy(acc.at[sl], r_a)

        @pl.when(cid == 0)
        def _():
            pltpu.sync_copy(r_a, out0.at[sl])

        @pl.when(cid == 1)
        def _():
            pltpu.sync_copy(r_a, out1.at[sl])


@functools.cache
def _get_sc_agg():
    return pl.kernel(
        _sc_agg_body,
        out_type=(jax.ShapeDtypeStruct((_NP, _D), _f32),
                  jax.ShapeDtypeStruct((_NP, _D), _f32)),
        mesh=plsc.VectorSubcoreMesh(core_axis_name="c", subcore_axis_name="s"),
        scratch_types=[
            pltpu.VMEM_SHARED((_NP, _D), _f32),
            pltpu.VMEM((2, _BLKC, _CHUNK), jnp.int32),
            pltpu.VMEM((2, _BLKC, _CHUNK), jnp.int32),
            pltpu.VMEM((_CHUNK, _D), _f32),
            pltpu.VMEM((_CHUNK, _D), _f32),
            pltpu.SemaphoreType.DMA,
            pltpu.SemaphoreType.DMA,
            pltpu.SemaphoreType.DMA,
            pltpu.SemaphoreType.DMA,
        ],
    )


def _mm_body(x_ref, w_ref, h_ref, cnt_ref):
    i = pl.program_id(0)
    xb = x_ref[...]
    h_ref[...] = jnp.dot(xb, w_ref[...], preferred_element_type=_f32)
    nz = jnp.logical_and(xb[:, 0:1] != 0.0, xb[:, 1:2] != 0.0)
    c = jnp.sum(nz.astype(_f32), axis=0, keepdims=True)

    @pl.when(i == 0)
    def _():
        cnt_ref[...] = c

    @pl.when(i != 0)
    def _():
        cnt_ref[...] = cnt_ref[...] + c


def _scale_body(d0_ref, d1_ref, h_ref, dinv_ref, hp_ref):
    deg = d0_ref[...] + d1_ref[...]
    dinv = lax.rsqrt(deg + 1.0)
    dinv_ref[...] = dinv
    hp_ref[...] = h_ref[...] * dinv


def _layer2_body(p0_ref, p1_ref, hp_ref, dinv_ref, b_ref, w_ref, out_ref):
    i = pl.program_id(0)
    dinv = dinv_ref[...]
    s = dinv * (p0_ref[...] + p1_ref[...] + hp_ref[...]) + b_ref[...]
    h = jnp.tanh(s)
    hp2 = jnp.dot(h, w_ref[...], preferred_element_type=_f32) * dinv
    rows = i * _BLK + lax.broadcasted_iota(jnp.int32, (_BLK, 1), 0)
    out_ref[...] = jnp.where(rows < _N, hp2, 0.0)


def _head_body(q0_ref, q1_ref, hp_ref, dinv_ref, b_ref, wfc_ref, bfc_ref,
               cnt_ref, out_ref, acc_ref):
    i = pl.program_id(0)
    s = dinv_ref[...] * (q0_ref[...] + q1_ref[...] + hp_ref[...]) + b_ref[...]
    h = jnp.tanh(s)
    rows = i * _BLK + lax.broadcasted_iota(jnp.int32, (_BLK, 1), 0)
    h = jnp.where(rows < _N, h, 0.0)
    part = jnp.sum(h, axis=0, keepdims=True)

    @pl.when(i == 0)
    def _():
        acc_ref[...] = part

    @pl.when(i != 0)
    def _():
        acc_ref[...] = acc_ref[...] + part

    @pl.when(i == _NBLK - 1)
    def _():
        pooled = acc_ref[...] * (1.0 / _N)
        vel = jnp.dot(pooled, wfc_ref[...], preferred_element_type=_f32)
        vel = vel + bfc_ref[...]
        robot = lax.broadcasted_iota(jnp.int32, (1, _ROBOTS * _OUT), 1) // _OUT
        mask = robot.astype(_f32) < cnt_ref[...]
        out_ref[...] = jnp.where(mask, vel, 0.0)


def _row_spec():
    return pl.BlockSpec((_BLK, _D), lambda i: (i, 0))


def _col_spec():
    return pl.BlockSpec((_BLK, 1), lambda i: (i, 0))


def _const_spec(shape):
    return pl.BlockSpec(shape, lambda i: (0,) * len(shape))


def _tc_mm(xp, w1):
    return pl.pallas_call(
        _mm_body,
        grid=(_NBLK,),
        in_specs=[_row_spec(), _const_spec((_D, _D))],
        out_specs=[_row_spec(), _const_spec((1, 1))],
        out_shape=[jax.ShapeDtypeStruct((_NP, _D), _f32),
                   jax.ShapeDtypeStruct((1, 1), _f32)],
    )(xp, w1)


def _tc_scale(d0, d1, h):
    return pl.pallas_call(
        _scale_body,
        grid=(_NBLK,),
        in_specs=[_col_spec(), _col_spec(), _row_spec()],
        out_specs=[_col_spec(), _row_spec()],
        out_shape=[jax.ShapeDtypeStruct((_NP, 1), _f32),
                   jax.ShapeDtypeStruct((_NP, _D), _f32)],
    )(d0, d1, h)


def _tc_layer2(p0, p1, hp, dinv, b1, w2):
    return pl.pallas_call(
        _layer2_body,
        grid=(_NBLK,),
        in_specs=[_row_spec(), _row_spec(), _row_spec(), _col_spec(),
                  _const_spec((1, _D)), _const_spec((_D, _D))],
        out_specs=_row_spec(),
        out_shape=jax.ShapeDtypeStruct((_NP, _D), _f32),
    )(p0, p1, hp, dinv, b1, w2)


def _tc_head(q0, q1, hp, dinv, b2, wfc, bfc, cnt):
    return pl.pallas_call(
        _head_body,
        grid=(_NBLK,),
        in_specs=[_row_spec(), _row_spec(), _row_spec(), _col_spec(),
                  _const_spec((1, _D)), _const_spec((_D, _ROBOTS * _OUT)),
                  _const_spec((1, _ROBOTS * _OUT)), _const_spec((1, 1))],
        out_specs=_const_spec((1, _ROBOTS * _OUT)),
        out_shape=jax.ShapeDtypeStruct((1, _ROBOTS * _OUT), _f32),
        scratch_shapes=[pltpu.VMEM((1, _D), _f32)],
    )(q0, q1, hp, dinv, b2, wfc, bfc, cnt)


def kernel(x, edge_index, batch, W1, b1, W2, b2, Wfc, bfc):
    del batch
    x = x.astype(_f32)
    ei = edge_index.astype(jnp.int32)
    pad_idx = _N + jnp.arange(_EPAD - _E, dtype=jnp.int32) % (_NP - _N)
    src = jnp.concatenate([ei[0], pad_idx]).reshape(_NW, _NBLKI, _BLKC, _CHUNK)
    dst = jnp.concatenate([ei[1], pad_idx]).reshape(_NW, _NBLKI, _BLKC, _CHUNK)
    xp = jnp.concatenate([x, jnp.zeros((_NP - _N, _D), _f32)], axis=0)
    zeros2d = jnp.zeros((_CHUNK, _D), _f32)

    d0, d1 = _get_sc_deg()(dst)
    h1raw, cnt = _tc_mm(xp, W1.astype(_f32))
    dinv, hp1 = _tc_scale(d0.reshape(_NP, 1), d1.reshape(_NP, 1), h1raw)
    p0, p1 = _get_sc_agg()(hp1, src, dst, zeros2d)
    hp2 = _tc_layer2(p0, p1, hp1, dinv, b1.astype(_f32).reshape(1, _D),
                     W2.astype(_f32))
    q0, q1 = _get_sc_agg()(hp2, src, dst, zeros2d)
    out = _tc_head(q0, q1, hp2, dinv, b2.astype(_f32).reshape(1, _D),
                   Wfc.astype(_f32), bfc.astype(_f32).reshape(1, _ROBOTS * _OUT),
                   cnt)
    return out.reshape(1, _ROBOTS, _OUT)


# speedup vs baseline: 28.2556x; 1.3327x over previous
import functools

import jax
import jax.numpy as jnp
from jax import lax
from jax.experimental import pallas as pl
from jax.experimental.pallas import tpu as pltpu
from jax.experimental.pallas import tpu_sc as plsc

_N = 10000
_D = 128
_E = 320000
_ROBOTS = 32
_OUT = 2

_NP = 10240
_NW = 32
_NT = 16
_CHUNK = 128
_CPT = 80
_BLKC = 16
_NBLKI = _CPT // _BLKC
_EPAD = _NW * _CPT * _CHUNK
_RPT = _NP // _NT

_BLK = 1024
_NBLK = _NP // _BLK

_f32 = jnp.float32


def _sc_deg_body(dst_hbm, out0, out1, deg_acc, dst_v, ones_v, buf_v):
    cid = lax.axis_index("c")
    sid = lax.axis_index("s")
    wid = cid * _NT + sid
    for k in range(_CHUNK // 16):
        ones_v[pl.ds(k * 16, 16)] = jnp.full((16,), 1.0, _f32)
    for k in range(_RPT // 16):
        buf_v[pl.ds(k * 16, 16)] = jnp.zeros((16,), _f32)
    pltpu.sync_copy(buf_v, deg_acc.at[pl.ds(sid * _RPT, _RPT)])
    plsc.subcore_barrier()

    for b in range(_NBLKI):
        pltpu.sync_copy(dst_hbm.at[wid].at[b], dst_v)

        def _step(k, carry):
            pltpu.sync_copy(ones_v, deg_acc.at[dst_v.at[k]], add=True)
            return carry

        lax.fori_loop(0, _BLKC, _step, 0)
    plsc.subcore_barrier()

    sl = pl.ds(sid * _RPT, _RPT)
    pltpu.sync_copy(deg_acc.at[sl], buf_v)

    @pl.when(cid == 0)
    def _():
        pltpu.sync_copy(buf_v, out0.at[sl])

    @pl.when(cid == 1)
    def _():
        pltpu.sync_copy(buf_v, out1.at[sl])


@functools.cache
def _get_sc_deg():
    return pl.kernel(
        _sc_deg_body,
        out_type=(jax.ShapeDtypeStruct((_NP,), _f32),
                  jax.ShapeDtypeStruct((_NP,), _f32)),
        mesh=plsc.VectorSubcoreMesh(core_axis_name="c", subcore_axis_name="s"),
        scratch_types=[
            pltpu.VMEM_SHARED((_NP,), _f32),
            pltpu.VMEM((_BLKC, _CHUNK), jnp.int32),
            pltpu.VMEM((_CHUNK,), _f32),
            pltpu.VMEM((_RPT,), _f32),
        ],
    )


def _sc_agg_body(hp_hbm, src_hbm, dst_hbm, zeros_hbm, out0, out1,
                 acc, src_v, dst_v, r_a, r_b, sem_a, sem_b, sem_i0, sem_i1):
    cid = lax.axis_index("c")
    sid = lax.axis_index("s")
    wid = cid * _NT + sid
    base = sid * _RPT
    pltpu.sync_copy(zeros_hbm, r_a)
    for k in range(_RPT // _CHUNK):
        pltpu.sync_copy(r_a, acc.at[pl.ds(base + k * _CHUNK, _CHUNK)])
    plsc.subcore_barrier()

    sems = (sem_i0, sem_i1)
    pltpu.async_copy(src_hbm.at[wid].at[0], src_v.at[0], sems[0])
    pltpu.async_copy(dst_hbm.at[wid].at[0], dst_v.at[0], sems[0])
    pltpu.async_copy(src_hbm.at[wid].at[1], src_v.at[1], sems[1])
    pltpu.async_copy(dst_hbm.at[wid].at[1], dst_v.at[1], sems[1])
    for b in range(_NBLKI):
        s = b % 2
        pltpu.make_async_copy(src_hbm.at[wid].at[b], src_v.at[s], sems[s]).wait()
        pltpu.make_async_copy(dst_hbm.at[wid].at[b], dst_v.at[s], sems[s]).wait()
        sv, dv = src_v.at[s], dst_v.at[s]
        pltpu.async_copy(hp_hbm.at[sv.at[0]], r_a, sem_a)
        pltpu.async_copy(hp_hbm.at[sv.at[1]], r_b, sem_b)

        def _step(i, carry):
            j = 2 * i
            pltpu.make_async_copy(hp_hbm.at[sv.at[j]], r_a, sem_a).wait()
            pltpu.sync_copy(r_a, acc.at[dv.at[j]], add=True)
            pltpu.async_copy(hp_hbm.at[sv.at[j + 2]], r_a, sem_a)
            pltpu.make_async_copy(hp_hbm.at[sv.at[j + 1]], r_b, sem_b).wait()
            pltpu.sync_copy(r_b, acc.at[dv.at[j + 1]], add=True)
            pltpu.async_copy(hp_hbm.at[sv.at[j + 3]], r_b, sem_b)
            return carry

        lax.fori_loop(0, _BLKC // 2 - 1, _step, 0)
        j = _BLKC - 2
        pltpu.make_async_copy(hp_hbm.at[sv.at[j]], r_a, sem_a).wait()
        pltpu.sync_copy(r_a, acc.at[dv.at[j]], add=True)
        pltpu.make_async_copy(hp_hbm.at[sv.at[j + 1]], r_b, sem_b).wait()
        pltpu.sync_copy(r_b, acc.at[dv.at[j + 1]], add=True)
        if b + 2 < _NBLKI:
            pltpu.async_copy(src_hbm.at[wid].at[b + 2], src_v.at[s], sems[s])
            pltpu.async_copy(dst_hbm.at[wid].at[b + 2], dst_v.at[s], sems[s])
    plsc.subcore_barrier()

    for k in range(_RPT // _CHUNK):
        sl = pl.ds(base + k * _CHUNK, _CHUNK)
        pltpu.sync_copy(acc.at[sl], r_a)

        @pl.when(cid == 0)
        def _():
            pltpu.sync_copy(r_a, out0.at[sl])

        @pl.when(cid == 1)
        def _():
            pltpu.sync_copy(r_a, out1.at[sl])


@functools.cache
def _get_sc_agg():
    return pl.kernel(
        _sc_agg_body,
        out_type=(jax.ShapeDtypeStruct((_NP, _D), _f32),
                  jax.ShapeDtypeStruct((_NP, _D), _f32)),
        mesh=plsc.VectorSubcoreMesh(core_axis_name="c", subcore_axis_name="s"),
        scratch_types=[
            pltpu.VMEM_SHARED((_NP, _D), _f32),
            pltpu.VMEM((2, _BLKC, _CHUNK), jnp.int32),
            pltpu.VMEM((2, _BLKC, _CHUNK), jnp.int32),
            pltpu.VMEM((_CHUNK, _D), _f32),
            pltpu.VMEM((_CHUNK, _D), _f32),
            pltpu.SemaphoreType.DMA,
            pltpu.SemaphoreType.DMA,
            pltpu.SemaphoreType.DMA,
            pltpu.SemaphoreType.DMA,
        ],
    )


def _mm_body(x_ref, w_ref, h_ref, cnt_ref):
    i = pl.program_id(0)
    xb = x_ref[...]
    h_ref[...] = jnp.dot(xb, w_ref[...], preferred_element_type=_f32)
    nz = jnp.logical_and(xb[:, 0:1] != 0.0, xb[:, 1:2] != 0.0)
    c = jnp.sum(nz.astype(_f32), axis=0, keepdims=True)

    @pl.when(i == 0)
    def _():
        cnt_ref[...] = c

    @pl.when(i != 0)
    def _():
        cnt_ref[...] = cnt_ref[...] + c


def _scale_body(d0_ref, d1_ref, h_ref, dinv_ref, hp_ref):
    deg = d0_ref[...] + d1_ref[...]
    dinv = lax.rsqrt(deg + 1.0)
    dinv_ref[...] = dinv
    hp_ref[...] = h_ref[...] * dinv


def _layer2_body(p0_ref, p1_ref, hp_ref, dinv_ref, b_ref, w_ref, out_ref):
    i = pl.program_id(0)
    dinv = dinv_ref[...]
    s = dinv * (p0_ref[...] + p1_ref[...] + hp_ref[...]) + b_ref[...]
    h = jnp.tanh(s)
    hp2 = jnp.dot(h, w_ref[...], preferred_element_type=_f32) * dinv
    rows = i * _BLK + lax.broadcasted_iota(jnp.int32, (_BLK, 1), 0)
    out_ref[...] = jnp.where(rows < _N, hp2, 0.0)


def _head_body(q0_ref, q1_ref, hp_ref, dinv_ref, b_ref, wfc_ref, bfc_ref,
               cnt_ref, out_ref, acc_ref):
    i = pl.program_id(0)
    s = dinv_ref[...] * (q0_ref[...] + q1_ref[...] + hp_ref[...]) + b_ref[...]
    h = jnp.tanh(s)
    rows = i * _BLK + lax.broadcasted_iota(jnp.int32, (_BLK, 1), 0)
    h = jnp.where(rows < _N, h, 0.0)
    part = jnp.sum(h, axis=0, keepdims=True)

    @pl.when(i == 0)
    def _():
        acc_ref[...] = part

    @pl.when(i != 0)
    def _():
        acc_ref[...] = acc_ref[...] + part

    @pl.when(i == _NBLK - 1)
    def _():
        pooled = acc_ref[...] * (1.0 / _N)
        vel = jnp.dot(pooled, wfc_ref[...], preferred_element_type=_f32)
        vel = vel + bfc_ref[...]
        robot = lax.broadcasted_iota(jnp.int32, (1, _ROBOTS * _OUT), 1) // _OUT
        mask = robot.astype(_f32) < cnt_ref[...]
        out_ref[...] = jnp.where(mask, vel, 0.0)


def _row_spec():
    return pl.BlockSpec((_BLK, _D), lambda i: (i, 0))


def _col_spec():
    return pl.BlockSpec((_BLK, 1), lambda i: (i, 0))


def _const_spec(shape):
    return pl.BlockSpec(shape, lambda i: (0,) * len(shape))


def _tc_mm(xp, w1):
    return pl.pallas_call(
        _mm_body,
        grid=(_NBLK,),
        in_specs=[_row_spec(), _const_spec((_D, _D))],
        out_specs=[_row_spec(), _const_spec((1, 1))],
        out_shape=[jax.ShapeDtypeStruct((_NP, _D), _f32),
                   jax.ShapeDtypeStruct((1, 1), _f32)],
    )(xp, w1)


def _tc_scale(d0, d1, h):
    return pl.pallas_call(
        _scale_body,
        grid=(_NBLK,),
        in_specs=[_col_spec(), _col_spec(), _row_spec()],
        out_specs=[_col_spec(), _row_spec()],
        out_shape=[jax.ShapeDtypeStruct((_NP, 1), _f32),
                   jax.ShapeDtypeStruct((_NP, _D), _f32)],
    )(d0, d1, h)


def _tc_layer2(p0, p1, hp, dinv, b1, w2):
    return pl.pallas_call(
        _layer2_body,
        grid=(_NBLK,),
        in_specs=[_row_spec(), _row_spec(), _row_spec(), _col_spec(),
                  _const_spec((1, _D)), _const_spec((_D, _D))],
        out_specs=_row_spec(),
        out_shape=jax.ShapeDtypeStruct((_NP, _D), _f32),
    )(p0, p1, hp, dinv, b1, w2)


def _tc_head(q0, q1, hp, dinv, b2, wfc, bfc, cnt):
    return pl.pallas_call(
        _head_body,
        grid=(_NBLK,),
        in_specs=[_row_spec(), _row_spec(), _row_spec(), _col_spec(),
                  _const_spec((1, _D)), _const_spec((_D, _ROBOTS * _OUT)),
                  _const_spec((1, _ROBOTS * _OUT)), _const_spec((1, 1))],
        out_specs=_const_spec((1, _ROBOTS * _OUT)),
        out_shape=jax.ShapeDtypeStruct((1, _ROBOTS * _OUT), _f32),
        scratch_shapes=[pltpu.VMEM((1, _D), _f32)],
    )(q0, q1, hp, dinv, b2, wfc, bfc, cnt)


def kernel(x, edge_index, batch, W1, b1, W2, b2, Wfc, bfc):
    del batch
    x = x.astype(_f32)
    ei = edge_index.astype(jnp.int32)
    pad_idx = _N + jnp.arange(_EPAD - _E, dtype=jnp.int32) % (_NP - _N)
    src = jnp.concatenate([ei[0], pad_idx]).reshape(_NW, _NBLKI, _BLKC, _CHUNK)
    dst = jnp.concatenate([ei[1], pad_idx]).reshape(_NW, _NBLKI, _BLKC, _CHUNK)
    xp = jnp.concatenate([x, jnp.zeros((_NP - _N, _D), _f32)], axis=0)
    zeros2d = jnp.zeros((_CHUNK, _D), _f32)

    d0, d1 = _get_sc_deg()(dst)
    h1raw, cnt = _tc_mm(xp, W1.astype(_f32))
    dinv, hp1 = _tc_scale(d0.reshape(_NP, 1), d1.reshape(_NP, 1), h1raw)
    p0, p1 = _get_sc_agg()(hp1, src, dst, zeros2d)
    hp2 = _tc_layer2(p0, p1, hp1, dinv, b1.astype(_f32).reshape(1, _D),
                     W2.astype(_f32))
    q0, q1 = _get_sc_agg()(hp2, src, dst, zeros2d)
    out = _tc_head(q0, q1, hp2, dinv, b2.astype(_f32).reshape(1, _D),
                   Wfc.astype(_f32), bfc.astype(_f32).reshape(1, _ROBOTS * _OUT),
                   cnt)
    return out.reshape(1, _ROBOTS, _OUT)


# speedup vs baseline: 28.6634x; 1.0144x over previous
import functools

import jax
import jax.numpy as jnp
import numpy as np
from jax import lax
from jax.experimental import pallas as pl
from jax.experimental.pallas import tpu as pltpu
from jax.experimental.pallas import tpu_sc as plsc

_N = 10000
_D = 128
_E = 320000
_ROBOTS = 32
_OUT = 2

_NP = 10240
_NW = 32
_NT = 16
_CHUNK = 64
_CPT = 160
_BLKC = 16
_NBLKI = _CPT // _BLKC
_NBUF = 4
_EPAD = _NW * _CPT * _CHUNK
_RPT = _NP // _NT

_BLK = 1024
_NBLK = _NP // _BLK

_f32 = jnp.float32


def _sc_deg_body(dst_hbm, out0, out1, deg_acc, dst_v, ones_v, buf_v):
    cid = lax.axis_index("c")
    sid = lax.axis_index("s")
    wid = cid * _NT + sid
    for k in range(_CHUNK // 16):
        ones_v[pl.ds(k * 16, 16)] = jnp.full((16,), 1.0, _f32)
    for k in range(_RPT // 16):
        buf_v[pl.ds(k * 16, 16)] = jnp.zeros((16,), _f32)
    pltpu.sync_copy(buf_v, deg_acc.at[pl.ds(sid * _RPT, _RPT)])
    plsc.subcore_barrier()

    for b in range(_NBLKI):
        pltpu.sync_copy(dst_hbm.at[wid].at[b], dst_v)

        def _step(k, carry):
            pltpu.sync_copy(ones_v, deg_acc.at[dst_v.at[k]], add=True)
            return carry

        lax.fori_loop(0, _BLKC, _step, 0)
    plsc.subcore_barrier()

    sl = pl.ds(sid * _RPT, _RPT)
    pltpu.sync_copy(deg_acc.at[sl], buf_v)

    @pl.when(cid == 0)
    def _():
        pltpu.sync_copy(buf_v, out0.at[sl])

    @pl.when(cid == 1)
    def _():
        pltpu.sync_copy(buf_v, out1.at[sl])


@functools.cache
def _get_sc_deg():
    return pl.kernel(
        _sc_deg_body,
        out_type=(jax.ShapeDtypeStruct((_NP,), _f32),
                  jax.ShapeDtypeStruct((_NP,), _f32)),
        mesh=plsc.VectorSubcoreMesh(core_axis_name="c", subcore_axis_name="s"),
        scratch_types=[
            pltpu.VMEM_SHARED((_NP,), _f32),
            pltpu.VMEM((_BLKC, _CHUNK), jnp.int32),
            pltpu.VMEM((_CHUNK,), _f32),
            pltpu.VMEM((_RPT,), _f32),
        ],
    )


def _sc_agg_body(hp_hbm, src_hbm, dst_hbm, zeros_hbm, out0, out1,
                 acc, src_v, dst_v, r0, r1, r2, r3,
                 g0, g1, g2, g3, s0, s1, s2, s3, i0, i1):
    bufs = (r0, r1, r2, r3)
    gsem = (g0, g1, g2, g3)
    ssem = (s0, s1, s2, s3)
    isem = (i0, i1)
    cid = lax.axis_index("c")
    sid = lax.axis_index("s")
    wid = cid * _NT + sid
    base = sid * _RPT
    pltpu.sync_copy(zeros_hbm, r0)
    for k in range(_RPT // _CHUNK):
        pltpu.sync_copy(r0, acc.at[pl.ds(base + k * _CHUNK, _CHUNK)])
    plsc.subcore_barrier()

    def sv(t):
        return src_v.at[(t // _BLKC) % 2].at[t % _BLKC]

    def dv(t):
        return dst_v.at[(t // _BLKC) % 2].at[t % _BLKC]

    def g_desc(t):
        return pltpu.make_async_copy(hp_hbm.at[sv(t)], bufs[t % _NBUF],
                                     gsem[t % _NBUF])

    def s_desc(t):
        return pltpu.make_async_copy(bufs[t % _NBUF], acc.at[dv(t)],
                                     ssem[t % _NBUF])

    pltpu.sync_copy(src_hbm.at[wid].at[0], src_v.at[0])
    pltpu.sync_copy(dst_hbm.at[wid].at[0], dst_v.at[0])
    pltpu.async_copy(src_hbm.at[wid].at[1], src_v.at[1], isem[1])
    pltpu.async_copy(dst_hbm.at[wid].at[1], dst_v.at[1], isem[1])
    for t in range(_NBUF - 1):
        g_desc(t).start()

    for t in range(_CPT):
        b, k = divmod(t, _BLKC)
        g_desc(t).wait()
        s_desc(t).start(add=True)
        tn = t + _NBUF - 1
        if tn < _CPT:
            if t >= 1:
                s_desc(t - 1).wait()
            if tn % _BLKC == 0:
                bb = tn // _BLKC
                pltpu.make_async_copy(src_hbm.at[wid].at[bb],
                                      src_v.at[bb % 2], isem[bb % 2]).wait()
                pltpu.make_async_copy(dst_hbm.at[wid].at[bb],
                                      dst_v.at[bb % 2], isem[bb % 2]).wait()
            g_desc(tn).start()
        if k == 3 and 1 <= b < _NBLKI - 1:
            pltpu.async_copy(src_hbm.at[wid].at[b + 1],
                             src_v.at[(b + 1) % 2], isem[(b + 1) % 2])
            pltpu.async_copy(dst_hbm.at[wid].at[b + 1],
                             dst_v.at[(b + 1) % 2], isem[(b + 1) % 2])
    for q in range(_CPT - 4, _CPT):
        s_desc(q).wait()
    plsc.subcore_barrier()

    for k in range(_RPT // _CHUNK):
        sl = pl.ds(base + k * _CHUNK, _CHUNK)
        pltpu.sync_copy(acc.at[sl], r0)

        @pl.when(cid == 0)
        def _():
            pltpu.sync_copy(r0, out0.at[sl])

        @pl.when(cid == 1)
        def _():
            pltpu.sync_copy(r0, out1.at[sl])


@functools.cache
def _get_sc_agg():
    return pl.kernel(
        _sc_agg_body,
        out_type=(jax.ShapeDtypeStruct((_NP, _D), _f32),
                  jax.ShapeDtypeStruct((_NP, _D), _f32)),
        mesh=plsc.VectorSubcoreMesh(core_axis_name="c", subcore_axis_name="s"),
        scratch_types=[
            pltpu.VMEM_SHARED((_NP, _D), _f32),
            pltpu.VMEM((2, _BLKC, _CHUNK), jnp.int32),
            pltpu.VMEM((2, _BLKC, _CHUNK), jnp.int32),
            pltpu.VMEM((_CHUNK, _D), _f32),
            pltpu.VMEM((_CHUNK, _D), _f32),
            pltpu.VMEM((_CHUNK, _D), _f32),
            pltpu.VMEM((_CHUNK, _D), _f32),
            pltpu.SemaphoreType.DMA,
            pltpu.SemaphoreType.DMA,
            pltpu.SemaphoreType.DMA,
            pltpu.SemaphoreType.DMA,
            pltpu.SemaphoreType.DMA,
            pltpu.SemaphoreType.DMA,
            pltpu.SemaphoreType.DMA,
            pltpu.SemaphoreType.DMA,
            pltpu.SemaphoreType.DMA,
            pltpu.SemaphoreType.DMA,
        ],
    )


def _mm_body(x_ref, w_ref, h_ref, cnt_ref):
    i = pl.program_id(0)
    xb = x_ref[...]
    h_ref[...] = jnp.dot(xb, w_ref[...], preferred_element_type=_f32)
    nz = jnp.logical_and(xb[:, 0:1] != 0.0, xb[:, 1:2] != 0.0)
    c = jnp.sum(nz.astype(_f32), axis=0, keepdims=True)

    @pl.when(i == 0)
    def _():
        cnt_ref[...] = c

    @pl.when(i != 0)
    def _():
        cnt_ref[...] = cnt_ref[...] + c


def _scale_body(d0_ref, d1_ref, h_ref, dinv_ref, hp_ref):
    deg = d0_ref[...] + d1_ref[...]
    dinv = lax.rsqrt(deg + 1.0)
    dinv_ref[...] = dinv
    hp_ref[...] = h_ref[...] * dinv


def _layer2_body(p0_ref, p1_ref, hp_ref, dinv_ref, b_ref, w_ref, out_ref):
    i = pl.program_id(0)
    dinv = dinv_ref[...]
    s = dinv * (p0_ref[...] + p1_ref[...] + hp_ref[...]) + b_ref[...]
    h = jnp.tanh(s)
    hp2 = jnp.dot(h, w_ref[...], preferred_element_type=_f32) * dinv
    rows = i * _BLK + lax.broadcasted_iota(jnp.int32, (_BLK, 1), 0)
    out_ref[...] = jnp.where(rows < _N, hp2, 0.0)


def _head_body(q0_ref, q1_ref, hp_ref, dinv_ref, b_ref, wfc_ref, bfc_ref,
               cnt_ref, out_ref, acc_ref):
    i = pl.program_id(0)
    s = dinv_ref[...] * (q0_ref[...] + q1_ref[...] + hp_ref[...]) + b_ref[...]
    h = jnp.tanh(s)
    rows = i * _BLK + lax.broadcasted_iota(jnp.int32, (_BLK, 1), 0)
    h = jnp.where(rows < _N, h, 0.0)
    part = jnp.sum(h, axis=0, keepdims=True)

    @pl.when(i == 0)
    def _():
        acc_ref[...] = part

    @pl.when(i != 0)
    def _():
        acc_ref[...] = acc_ref[...] + part

    @pl.when(i == _NBLK - 1)
    def _():
        pooled = acc_ref[...] * (1.0 / _N)
        vel = jnp.dot(pooled, wfc_ref[...], preferred_element_type=_f32)
        vel = vel + bfc_ref[...]
        robot = lax.broadcasted_iota(jnp.int32, (1, _ROBOTS * _OUT), 1) // _OUT
        mask = robot.astype(_f32) < cnt_ref[...]
        out_ref[...] = jnp.where(mask, vel, 0.0)


def _row_spec():
    return pl.BlockSpec((_BLK, _D), lambda i: (i, 0))


def _col_spec():
    return pl.BlockSpec((_BLK, 1), lambda i: (i, 0))


def _const_spec(shape):
    return pl.BlockSpec(shape, lambda i: (0,) * len(shape))


def _tc_mm(xp, w1):
    return pl.pallas_call(
        _mm_body,
        grid=(_NBLK,),
        in_specs=[_row_spec(), _const_spec((_D, _D))],
        out_specs=[_row_spec(), _const_spec((1, 1))],
        out_shape=[jax.ShapeDtypeStruct((_NP, _D), _f32),
                   jax.ShapeDtypeStruct((1, 1), _f32)],
    )(xp, w1)


def _tc_scale(d0, d1, h):
    return pl.pallas_call(
        _scale_body,
        grid=(_NBLK,),
        in_specs=[_col_spec(), _col_spec(), _row_spec()],
        out_specs=[_col_spec(), _row_spec()],
        out_shape=[jax.ShapeDtypeStruct((_NP, 1), _f32),
                   jax.ShapeDtypeStruct((_NP, _D), _f32)],
    )(d0, d1, h)


def _tc_layer2(p0, p1, hp, dinv, b1, w2):
    return pl.pallas_call(
        _layer2_body,
        grid=(_NBLK,),
        in_specs=[_row_spec(), _row_spec(), _row_spec(), _col_spec(),
                  _const_spec((1, _D)), _const_spec((_D, _D))],
        out_specs=_row_spec(),
        out_shape=jax.ShapeDtypeStruct((_NP, _D), _f32),
    )(p0, p1, hp, dinv, b1, w2)


def _tc_head(q0, q1, hp, dinv, b2, wfc, bfc, cnt):
    return pl.pallas_call(
        _head_body,
        grid=(_NBLK,),
        in_specs=[_row_spec(), _row_spec(), _row_spec(), _col_spec(),
                  _const_spec((1, _D)), _const_spec((_D, _ROBOTS * _OUT)),
                  _const_spec((1, _ROBOTS * _OUT)), _const_spec((1, 1))],
        out_specs=_const_spec((1, _ROBOTS * _OUT)),
        out_shape=jax.ShapeDtypeStruct((1, _ROBOTS * _OUT), _f32),
        scratch_shapes=[pltpu.VMEM((1, _D), _f32)],
    )(q0, q1, hp, dinv, b2, wfc, bfc, cnt)


_PAD_IDX = (_N + np.arange(_EPAD - _E, dtype=np.int32) % (_NP - _N))


def kernel(x, edge_index, batch, W1, b1, W2, b2, Wfc, bfc):
    del batch
    x = x.astype(_f32)
    ei = edge_index.astype(jnp.int32)
    pad_idx = jnp.asarray(_PAD_IDX)
    src = jnp.concatenate([ei[0], pad_idx]).reshape(_NW, _NBLKI, _BLKC, _CHUNK)
    dst = jnp.concatenate([ei[1], pad_idx]).reshape(_NW, _NBLKI, _BLKC, _CHUNK)
    xp = jnp.concatenate([x, jnp.zeros((_NP - _N, _D), _f32)], axis=0)
    zeros2d = jnp.zeros((_CHUNK, _D), _f32)

    d0, d1 = _get_sc_deg()(dst)
    h1raw, cnt = _tc_mm(xp, W1.astype(_f32))
    dinv, hp1 = _tc_scale(d0.reshape(_NP, 1), d1.reshape(_NP, 1), h1raw)
    p0, p1 = _get_sc_agg()(hp1, src, dst, zeros2d)
    hp2 = _tc_layer2(p0, p1, hp1, dinv, b1.astype(_f32).reshape(1, _D),
                     W2.astype(_f32))
    q0, q1 = _get_sc_agg()(hp2, src, dst, zeros2d)
    out = _tc_head(q0, q1, hp2, dinv, b2.astype(_f32).reshape(1, _D),
                   Wfc.astype(_f32), bfc.astype(_f32).reshape(1, _ROBOTS * _OUT),
                   cnt)
    return out.reshape(1, _ROBOTS, _OUT)


# speedup vs baseline: 28.7911x; 1.0045x over previous
import functools

import jax
import jax.numpy as jnp
import numpy as np
from jax import lax
from jax.experimental import pallas as pl
from jax.experimental.pallas import tpu as pltpu
from jax.experimental.pallas import tpu_sc as plsc

_N = 10000
_D = 128
_E = 320000
_ROBOTS = 32
_OUT = 2

_NP = 10240
_NW = 32
_NT = 16
_CHUNK = 64
_CPT = 160
_BLKC = 16
_NBLKI = _CPT // _BLKC
_NBUF = 4
_EPAD = _NW * _CPT * _CHUNK
_RPT = _NP // _NT

_BLK = 1024
_NBLK = _NP // _BLK

_f32 = jnp.float32


def _sc_deg_body(dst_hbm, out0, out1, deg_acc, dst_v, ones_v, buf_v):
    cid = lax.axis_index("c")
    sid = lax.axis_index("s")
    wid = cid * _NT + sid
    for k in range(_CHUNK // 16):
        ones_v[pl.ds(k * 16, 16)] = jnp.full((16,), 1.0, _f32)
    for k in range(_RPT // 16):
        buf_v[pl.ds(k * 16, 16)] = jnp.zeros((16,), _f32)
    pltpu.sync_copy(buf_v, deg_acc.at[pl.ds(sid * _RPT, _RPT)])
    plsc.subcore_barrier()

    for b in range(_NBLKI):
        pltpu.sync_copy(dst_hbm.at[wid].at[b], dst_v)

        def _step(k, carry):
            pltpu.sync_copy(ones_v, deg_acc.at[dst_v.at[k]], add=True)
            return carry

        lax.fori_loop(0, _BLKC, _step, 0)
    plsc.subcore_barrier()

    sl = pl.ds(sid * _RPT, _RPT)
    pltpu.sync_copy(deg_acc.at[sl], buf_v)

    @pl.when(cid == 0)
    def _():
        pltpu.sync_copy(buf_v, out0.at[sl])

    @pl.when(cid == 1)
    def _():
        pltpu.sync_copy(buf_v, out1.at[sl])


@functools.cache
def _get_sc_deg():
    return pl.kernel(
        _sc_deg_body,
        out_type=(jax.ShapeDtypeStruct((_NP,), _f32),
                  jax.ShapeDtypeStruct((_NP,), _f32)),
        mesh=plsc.VectorSubcoreMesh(core_axis_name="c", subcore_axis_name="s"),
        scratch_types=[
            pltpu.VMEM_SHARED((_NP,), _f32),
            pltpu.VMEM((_BLKC, _CHUNK), jnp.int32),
            pltpu.VMEM((_CHUNK,), _f32),
            pltpu.VMEM((_RPT,), _f32),
        ],
    )


def _sc_agg_body(hp_hbm, src_hbm, dst_hbm, zeros_hbm, out0, out1,
                 acc, src_v, dst_v, r0, r1, r2, r3,
                 g0, g1, g2, g3, s0, s1, s2, s3, i0, i1):
    bufs = (r0, r1, r2, r3)
    gsem = (g0, g1, g2, g3)
    ssem = (s0, s1, s2, s3)
    isem = (i0, i1)
    cid = lax.axis_index("c")
    sid = lax.axis_index("s")
    wid = cid * _NT + sid
    base = sid * _RPT
    pltpu.sync_copy(zeros_hbm, r0)
    for k in range(_RPT // _CHUNK):
        pltpu.sync_copy(r0, acc.at[pl.ds(base + k * _CHUNK, _CHUNK)])
    plsc.subcore_barrier()

    def sv(t):
        return src_v.at[(t // _BLKC) % 2].at[t % _BLKC]

    def dv(t):
        return dst_v.at[(t // _BLKC) % 2].at[t % _BLKC]

    def g_desc(t):
        return pltpu.make_async_copy(hp_hbm.at[sv(t)], bufs[t % _NBUF],
                                     gsem[t % _NBUF])

    def s_desc(t):
        return pltpu.make_async_copy(bufs[t % _NBUF], acc.at[dv(t)],
                                     ssem[t % _NBUF])

    pltpu.sync_copy(src_hbm.at[wid].at[0], src_v.at[0])
    pltpu.sync_copy(dst_hbm.at[wid].at[0], dst_v.at[0])
    pltpu.async_copy(src_hbm.at[wid].at[1], src_v.at[1], isem[1])
    pltpu.async_copy(dst_hbm.at[wid].at[1], dst_v.at[1], isem[1])
    for t in range(_NBUF - 1):
        g_desc(t).start()

    for t in range(_CPT):
        b, k = divmod(t, _BLKC)
        g_desc(t).wait()
        s_desc(t).start(add=True)
        tn = t + _NBUF - 1
        if tn < _CPT:
            if t >= 1:
                s_desc(t - 1).wait()
            if tn % _BLKC == 0:
                bb = tn // _BLKC
                pltpu.make_async_copy(src_hbm.at[wid].at[bb],
                                      src_v.at[bb % 2], isem[bb % 2]).wait()
                pltpu.make_async_copy(dst_hbm.at[wid].at[bb],
                                      dst_v.at[bb % 2], isem[bb % 2]).wait()
            g_desc(tn).start()
        if k == 3 and 1 <= b < _NBLKI - 1:
            pltpu.async_copy(src_hbm.at[wid].at[b + 1],
                             src_v.at[(b + 1) % 2], isem[(b + 1) % 2])
            pltpu.async_copy(dst_hbm.at[wid].at[b + 1],
                             dst_v.at[(b + 1) % 2], isem[(b + 1) % 2])
    for q in range(_CPT - 4, _CPT):
        s_desc(q).wait()
    plsc.subcore_barrier()

    for k in range(_RPT // _CHUNK):
        sl = pl.ds(base + k * _CHUNK, _CHUNK)
        pltpu.sync_copy(acc.at[sl], r0)

        @pl.when(cid == 0)
        def _():
            pltpu.sync_copy(r0, out0.at[sl])

        @pl.when(cid == 1)
        def _():
            pltpu.sync_copy(r0, out1.at[sl])


@functools.cache
def _get_sc_agg():
    return pl.kernel(
        _sc_agg_body,
        out_type=(jax.ShapeDtypeStruct((_NP, _D), _f32),
                  jax.ShapeDtypeStruct((_NP, _D), _f32)),
        mesh=plsc.VectorSubcoreMesh(core_axis_name="c", subcore_axis_name="s"),
        scratch_types=[
            pltpu.VMEM_SHARED((_NP, _D), _f32),
            pltpu.VMEM((2, _BLKC, _CHUNK), jnp.int32),
            pltpu.VMEM((2, _BLKC, _CHUNK), jnp.int32),
            pltpu.VMEM((_CHUNK, _D), _f32),
            pltpu.VMEM((_CHUNK, _D), _f32),
            pltpu.VMEM((_CHUNK, _D), _f32),
            pltpu.VMEM((_CHUNK, _D), _f32),
            pltpu.SemaphoreType.DMA,
            pltpu.SemaphoreType.DMA,
            pltpu.SemaphoreType.DMA,
            pltpu.SemaphoreType.DMA,
            pltpu.SemaphoreType.DMA,
            pltpu.SemaphoreType.DMA,
            pltpu.SemaphoreType.DMA,
            pltpu.SemaphoreType.DMA,
            pltpu.SemaphoreType.DMA,
            pltpu.SemaphoreType.DMA,
        ],
    )


def _mm_body(x_ref, w_ref, d0_ref, d1_ref, dinv_ref, hp_ref, cnt_ref):
    i = pl.program_id(0)
    xb = x_ref[...]
    h = jnp.dot(xb, w_ref[...], preferred_element_type=_f32)
    dinv = lax.rsqrt(d0_ref[...] + d1_ref[...] + 1.0)
    dinv_ref[...] = dinv
    hp_ref[...] = h * dinv
    nz = jnp.logical_and(xb[:, 0:1] != 0.0, xb[:, 1:2] != 0.0)
    c = jnp.sum(nz.astype(_f32), axis=0, keepdims=True)

    @pl.when(i == 0)
    def _():
        cnt_ref[...] = c

    @pl.when(i != 0)
    def _():
        cnt_ref[...] = cnt_ref[...] + c


def _layer2_body(p0_ref, p1_ref, hp_ref, dinv_ref, b_ref, w_ref, out_ref):
    i = pl.program_id(0)
    dinv = dinv_ref[...]
    s = dinv * (p0_ref[...] + p1_ref[...] + hp_ref[...]) + b_ref[...]
    h = jnp.tanh(s)
    hp2 = jnp.dot(h, w_ref[...], preferred_element_type=_f32) * dinv
    rows = i * _BLK + lax.broadcasted_iota(jnp.int32, (_BLK, 1), 0)
    out_ref[...] = jnp.where(rows < _N, hp2, 0.0)


def _head_body(q0_ref, q1_ref, hp_ref, dinv_ref, b_ref, wfc_ref, bfc_ref,
               cnt_ref, out_ref, acc_ref):
    i = pl.program_id(0)
    s = dinv_ref[...] * (q0_ref[...] + q1_ref[...] + hp_ref[...]) + b_ref[...]
    h = jnp.tanh(s)
    rows = i * _BLK + lax.broadcasted_iota(jnp.int32, (_BLK, 1), 0)
    h = jnp.where(rows < _N, h, 0.0)
    part = jnp.sum(h, axis=0, keepdims=True)

    @pl.when(i == 0)
    def _():
        acc_ref[...] = part

    @pl.when(i != 0)
    def _():
        acc_ref[...] = acc_ref[...] + part

    @pl.when(i == _NBLK - 1)
    def _():
        pooled = acc_ref[...] * (1.0 / _N)
        vel = jnp.dot(pooled, wfc_ref[...], preferred_element_type=_f32)
        vel = vel + bfc_ref[...]
        robot = lax.broadcasted_iota(jnp.int32, (1, _ROBOTS * _OUT), 1) // _OUT
        mask = robot.astype(_f32) < cnt_ref[...]
        out_ref[...] = jnp.where(mask, vel, 0.0)


def _row_spec():
    return pl.BlockSpec((_BLK, _D), lambda i: (i, 0))


def _col_spec():
    return pl.BlockSpec((_BLK, 1), lambda i: (i, 0))


def _const_spec(shape):
    return pl.BlockSpec(shape, lambda i: (0,) * len(shape))


def _tc_mm(xp, w1, d0, d1):
    return pl.pallas_call(
        _mm_body,
        grid=(_NBLK,),
        in_specs=[_row_spec(), _const_spec((_D, _D)), _col_spec(), _col_spec()],
        out_specs=[_col_spec(), _row_spec(), _const_spec((1, 1))],
        out_shape=[jax.ShapeDtypeStruct((_NP, 1), _f32),
                   jax.ShapeDtypeStruct((_NP, _D), _f32),
                   jax.ShapeDtypeStruct((1, 1), _f32)],
    )(xp, w1, d0, d1)


def _tc_layer2(p0, p1, hp, dinv, b1, w2):
    return pl.pallas_call(
        _layer2_body,
        grid=(_NBLK,),
        in_specs=[_row_spec(), _row_spec(), _row_spec(), _col_spec(),
                  _const_spec((1, _D)), _const_spec((_D, _D))],
        out_specs=_row_spec(),
        out_shape=jax.ShapeDtypeStruct((_NP, _D), _f32),
    )(p0, p1, hp, dinv, b1, w2)


def _tc_head(q0, q1, hp, dinv, b2, wfc, bfc, cnt):
    return pl.pallas_call(
        _head_body,
        grid=(_NBLK,),
        in_specs=[_row_spec(), _row_spec(), _row_spec(), _col_spec(),
                  _const_spec((1, _D)), _const_spec((_D, _ROBOTS * _OUT)),
                  _const_spec((1, _ROBOTS * _OUT)), _const_spec((1, 1))],
        out_specs=_const_spec((1, _ROBOTS * _OUT)),
        out_shape=jax.ShapeDtypeStruct((1, _ROBOTS * _OUT), _f32),
        scratch_shapes=[pltpu.VMEM((1, _D), _f32)],
    )(q0, q1, hp, dinv, b2, wfc, bfc, cnt)


_PAD_IDX = (_N + np.arange(_EPAD - _E, dtype=np.int32) % (_NP - _N))
_PAD_2D = np.stack([_PAD_IDX, _PAD_IDX])


def kernel(x, edge_index, batch, W1, b1, W2, b2, Wfc, bfc):
    del batch
    x = x.astype(_f32)
    ei = edge_index.astype(jnp.int32)
    full = jnp.concatenate([ei, jnp.asarray(_PAD_2D)], axis=1)
    src = full[0].reshape(_NW, _NBLKI, _BLKC, _CHUNK)
    dst = full[1].reshape(_NW, _NBLKI, _BLKC, _CHUNK)
    xp = jnp.concatenate([x, jnp.zeros((_NP - _N, _D), _f32)], axis=0)
    zeros2d = jnp.zeros((_CHUNK, _D), _f32)

    d0, d1 = _get_sc_deg()(dst)
    dinv, hp1, cnt = _tc_mm(xp, W1.astype(_f32),
                            d0.reshape(_NP, 1), d1.reshape(_NP, 1))
    p0, p1 = _get_sc_agg()(hp1, src, dst, zeros2d)
    hp2 = _tc_layer2(p0, p1, hp1, dinv, b1.astype(_f32).reshape(1, _D),
                     W2.astype(_f32))
    q0, q1 = _get_sc_agg()(hp2, src, dst, zeros2d)
    out = _tc_head(q0, q1, hp2, dinv, b2.astype(_f32).reshape(1, _D),
                   Wfc.astype(_f32), bfc.astype(_f32).reshape(1, _ROBOTS * _OUT),
                   cnt)
    return out.reshape(1, _ROBOTS, _OUT)
